# Initial kernel scaffold; baseline (speedup 1.0000x reference)
import jax
import jax.numpy as jnp
from jax.experimental import pallas as pl


def kernel(points, norms, valid, index_map):
    raise NotImplementedError("write your pallas kernel here")


# speedup vs baseline: 2.1541x; 2.1541x over previous
import functools

import numpy as np
import jax
import jax.numpy as jnp
from jax import lax
from jax.experimental import pallas as pl
from jax.experimental.pallas import tpu as pltpu
from jax.experimental.pallas import tpu_sc as plsc

jax.config.update("jax_enable_x64", True)

G = 256
NW = 32
ROWS_W = G // NW
NODES_W = ROWS_W * G
EPR = 4 * (G - 1) + 1
E_W = ROWS_W * EPR
A_W = ROWS_W * 2 * (G - 1)
N_EDGES = (G - 1) * EPR + (G - 1)
N_TRIS = 2 * (G - 1) * (G - 1)
PIXW = 1024 * 3


def _build_constants():
    r = np.arange(G - 1)
    c = np.arange(G - 1)
    rr, cc = np.meshgrid(r, c, indexing="ij")
    n = rr * G + cc
    e4 = np.stack([
        np.stack([n, n + 1], -1),
        np.stack([n, n + G + 1], -1),
        np.stack([n, n + G], -1),
        np.stack([n + 1, n + G], -1),
    ], axis=2)
    last_col = np.stack([r * G + (G - 1), r * G + (G - 1) + G], -1).reshape(G - 1, 1, 2)
    rows = np.concatenate([e4.reshape(G - 1, -1, 2), last_col], axis=1)
    lastrow_n = (G - 1) * G + c
    last = np.stack([lastrow_n, lastrow_n + 1], -1)
    ei = np.concatenate([rows.reshape(-1, 2), last], axis=0).T.astype(np.int64)

    t0 = np.stack([n, n + 1, n + G + 1], axis=-1)
    t1 = np.stack([n, n + G + 1, n + G], axis=-1)
    tri = np.stack([t0, t1], axis=2).reshape(-1, 3).T.astype(np.int64)

    ra = np.arange(G)[:, None] * np.ones((1, G), np.int64)
    ca = np.ones((G, 1), np.int64) * np.arange(G)[None, :]
    specs = [
        (ra, 4 * ca, ca < G - 1),
        (ra, 4 * ca - 4, ca > 0),
        (ra, 4 * ca + 2, ra < G - 1),
        (ra - 1, 4 * ca + 2, ra > 0),
        (ra, 4 * ca + 1, (ra < G - 1) & (ca < G - 1)),
        (ra - 1, 4 * ca - 3, (ra > 0) & (ca > 0)),
        (ra, 4 * ca - 1, (ra < G - 1) & (ca > 0)),
        (ra - 1, 4 * ca + 3, (ra > 0) & (ca < G - 1)),
    ]
    base_row = 8 * (ra // 8) - 1
    pad = ((ra % 8) + 1) * 1024 + 1023
    inc = np.zeros((8, G, G), np.int64)
    deg = np.zeros((G, G), np.int64)
    for s, (er, off, ok) in enumerate(specs):
        lid = (er - base_row) * 1024 + off
        inc[s] = np.where(ok, lid, pad)
        deg += ok
    invdeg = (1.0 / deg).astype(np.float32)
    return ei, tri, inc.reshape(-1).astype(np.int32), invdeg.reshape(-1)


_EI_NP, _TRI_NP, _INC_NP, _INVDEG_NP = _build_constants()


def _rsqrt(q):
    i = plsc.bitcast(q, jnp.int32)
    i = jnp.int32(0x5F3759DF) - (i >> 1)
    r = plsc.bitcast(i, jnp.float32)
    hq = 0.5 * q
    r = r * (1.5 - hq * r * r)
    r = r * (1.5 - hq * r * r)
    return r


_mesh = plsc.VectorSubcoreMesh(core_axis_name="c", subcore_axis_name="s")


@functools.partial(
    pl.kernel,
    mesh=_mesh,
    compiler_params=pltpu.CompilerParams(needs_layout_passes=False),
    out_type=[
        jax.ShapeDtypeStruct((G * G * 3,), jnp.float32),
        jax.ShapeDtypeStruct((G * G * 3,), jnp.float32),
        jax.ShapeDtypeStruct((G * G,), jnp.float32),
        jax.ShapeDtypeStruct((NW * E_W,), jnp.float32),
        jax.ShapeDtypeStruct((NW * A_W,), jnp.float32),
    ],
    scratch_types=[
        pltpu.VMEM((2560,), jnp.int32),
        pltpu.VMEM((10 * PIXW,), jnp.float32),
        pltpu.VMEM((9216,), jnp.float32),
        pltpu.VMEM((E_W,), jnp.float32),
        pltpu.VMEM((A_W,), jnp.float32),
        pltpu.VMEM((16384,), jnp.int32),
        pltpu.VMEM((NODES_W,), jnp.float32),
        pltpu.VMEM((NODES_W,), jnp.float32),
        pltpu.VMEM((3 * NODES_W,), jnp.float32),
        pltpu.SemaphoreType.DMA,
    ],
)
def _sc_graph(points, norms, vp, inc, invdeg,
              edp_out, edn_out, radii_out, lens_out, areas_out,
              idx_f, slab, lpad, lens_st, areas_st,
              inc_v, invdeg_v, radii_st, ed_st, sem):
    i32 = jnp.int32
    wid = lax.axis_index("s") * 2 + lax.axis_index("c")
    iota = lax.iota(jnp.int32, 16)
    big = i32(1 << 30)

    pltpu.sync_copy(vp.at[pl.ds(NODES_W * wid, 2560)], idx_f)

    def row_id(l):
        vec = idx_f[pl.ds(i32(256 * l), 16)]
        return jnp.min(jnp.where(iota == 0, vec, big)) >> 10

    def pull_rows(src, rows):
        hs = []
        for l in rows:
            rid = row_id(l)
            hs.append(pltpu.async_copy(
                src.at[pl.ds(rid * PIXW, PIXW)],
                slab.at[pl.ds(i32(PIXW * l), PIXW)], sem))
        for h in hs:
            h.wait()

    pull_rows(points, range(10))

    for s in range(8):
        pltpu.sync_copy(inc.at[pl.ds(65536 * s + NODES_W * wid, 2048)],
                        inc_v.at[pl.ds(i32(2048 * s), 2048)])
    pltpu.sync_copy(invdeg.at[pl.ds(NODES_W * wid, NODES_W)], invdeg_v)

    def ldp(nidx):
        nf = plsc.load_gather(idx_f, [nidx])
        li = PIXW * (nidx >> 8) + 3 * (nf & i32(1023))
        return (plsc.load_gather(slab, [li]),
                plsc.load_gather(slab, [li + 1]),
                plsc.load_gather(slab, [li + 2]))

    def ed_pass(out_ref):
        def ebody(m, carry):
            n = 256 + 16 * m + iota
            x, y, z = ldp(n)
            o = 3 * (16 * m + iota)
            plsc.store_scatter(ed_st, [o], x)
            plsc.store_scatter(ed_st, [o + 1], y)
            plsc.store_scatter(ed_st, [o + 2], z)
            return carry
        lax.fori_loop(i32(0), i32(NODES_W // 16), ebody, i32(0))
        pltpu.sync_copy(ed_st, out_ref.at[pl.ds(3 * NODES_W * wid, 3 * NODES_W)])

    ed_pass(edp_out)

    def dist2(ax, ay, az, bx, by, bz):
        dx = bx - ax
        dy = by - ay
        dz = bz - az
        return dx * dx + dy * dy + dz * dz, (dx, dy, dz)

    def body(it, carry):
        l = lax.div(it, i32(16))
        k = lax.rem(it, i32(16))
        r_glob = 8 * wid - 1 + l
        base = 256 * l + 16 * k
        cvec = 16 * k + iota
        ax, ay, az = ldp(base + iota)
        bx, by, bz = ldp(base + 1 + iota)
        cx, cy, cz = ldp(jnp.minimum(base + 257 + iota, 2559))
        dx_, dy_, dz_ = ldp(base + 256 + iota)

        q0, dAB = dist2(ax, ay, az, bx, by, bz)
        q1, dAC = dist2(ax, ay, az, cx, cy, cz)
        q2, dAD = dist2(ax, ay, az, dx_, dy_, dz_)
        q3, _ = dist2(bx, by, bz, dx_, dy_, dz_)
        Ls = [q * _rsqrt(q) for q in (q0, q1, q2, q3)]

        not_lastcol = cvec < G - 1
        row_ok = jnp.broadcast_to(r_glob < G - 1, (16,))
        is_last = jnp.broadcast_to(r_glob == G - 1, (16,))
        lpos = jnp.broadcast_to(l > 0, (16,))
        valids = [not_lastcol, row_ok & not_lastcol, row_ok, row_ok & not_lastcol]

        for t in range(4):
            plsc.store_scatter(lpad, [1024 * l + 4 * cvec + t],
                               jnp.where(valids[t], Ls[t], 0.0))
            if t == 0:
                off = jnp.where(is_last, cvec, 4 * cvec)
                m = not_lastcol & lpos
            else:
                off = jnp.where(cvec == G - 1, 1020, 4 * cvec + t)
                m = valids[t] & lpos
            sidx = jnp.clip(EPR * (l - 1) + off, 0, E_W - 1)
            plsc.store_scatter(lens_st, [sidx], Ls[t], mask=m)

        am = not_lastcol & row_ok & lpos
        ux, uy, uz = dAB
        vx, vy, vz = dAC
        wx, wy, wz = dAD
        cr0x = uy * vz - uz * vy
        cr0y = uz * vx - ux * vz
        cr0z = ux * vy - uy * vx
        cr1x = vy * wz - vz * wy
        cr1y = vz * wx - vx * wz
        cr1z = vx * wy - vy * wx
        qa0 = cr0x * cr0x + cr0y * cr0y + cr0z * cr0z + 1e-13
        qa1 = cr1x * cr1x + cr1y * cr1y + cr1z * cr1z + 1e-13
        a0 = 0.5 * qa0 * _rsqrt(qa0)
        a1 = 0.5 * qa1 * _rsqrt(qa1)
        abase = jnp.clip(510 * (l - 1) + 2 * cvec, 0, A_W - 2)
        plsc.store_scatter(areas_st, [abase], a0, mask=am)
        plsc.store_scatter(areas_st, [abase + 1], a1, mask=am)
        return carry

    lax.fori_loop(i32(0), i32(9 * 16), body, i32(0))

    pull_rows(norms, range(1, 9))
    ed_pass(edn_out)

    def rbody(m, carry):
        base = 16 * m
        acc = plsc.load_gather(lpad, [inc_v[pl.ds(base, 16)]])
        for s in range(1, 8):
            acc = acc + plsc.load_gather(lpad, [inc_v[pl.ds(2048 * s + base, 16)]])
        radii_st[pl.ds(base, 16)] = acc * invdeg_v[pl.ds(base, 16)]
        return carry

    lax.fori_loop(i32(0), i32(NODES_W // 16), rbody, i32(0))

    pltpu.sync_copy(radii_st, radii_out.at[pl.ds(NODES_W * wid, NODES_W)])
    pltpu.sync_copy(lens_st, lens_out.at[pl.ds(E_W * wid, E_W)])
    pltpu.sync_copy(areas_st, areas_out.at[pl.ds(A_W * wid, A_W)])


def kernel(points, norms, valid, index_map):
    del valid
    vp = index_map[0:4 * (G - 1) + 1:4, 0:4 * (G - 1) + 1:4]
    vp = vp.reshape(-1).astype(jnp.int32)
    vp_pad = jnp.concatenate([vp[:G], vp, vp[-G:]])
    edp, edn, radii, lens_p, areas_p = _sc_graph(
        points.reshape(-1), norms.reshape(-1), vp_pad,
        jnp.asarray(_INC_NP), jnp.asarray(_INVDEG_NP))
    return (edp.reshape(G * G, 3), edn.reshape(G * G, 3), radii,
            jnp.asarray(_EI_NP), lens_p[:N_EDGES],
            jnp.asarray(_TRI_NP), areas_p[:N_TRIS])
